# Initial kernel scaffold; baseline (speedup 1.0000x reference)
import functools

import jax
import jax.numpy as jnp
from jax import lax
from jax.experimental import pallas as pl
from jax.experimental.pallas import tpu as pltpu
from jax.experimental.pallas import tpu_sc as plsc

N = 2048
E = 64
D = 768
F = 512
SF = 1024
NPAIR = 2 * N
BLK = 128
NB = (NPAIR + E * BLK) // BLK
P = NB * BLK
BE_LEN = 128
NEG = -1e30

_INTERPRET = False


def _router_body(x_ref, wr_ref, pos_ref, w_ref, be_ref, aux_ref):
    x = x_ref[...]
    wr = wr_ref[...]
    logits = lax.dot_general(x, wr, (((1,), (1,)), ((), ())),
                             preferred_element_type=jnp.float32)
    iota_e = lax.broadcasted_iota(jnp.int32, (N, E), 1)
    m1 = jnp.max(logits, axis=1, keepdims=True)
    a1 = jnp.min(jnp.where(logits == m1, iota_e, E), axis=1, keepdims=True)
    masked = jnp.where(iota_e == a1, NEG, logits)
    m2 = jnp.max(masked, axis=1, keepdims=True)
    a2 = jnp.min(jnp.where(masked == m2, iota_e, E), axis=1, keepdims=True)
    s = jnp.exp(m2 - m1)
    w_ref[...] = jnp.concatenate([1.0 / (1.0 + s), s / (1.0 + s)], axis=0)

    sm = jnp.exp(logits - m1)
    sm = sm / jnp.sum(sm, axis=1, keepdims=True)
    p_mean = jnp.sum(sm, axis=0, keepdims=True) * (1.0 / N)

    e_pair = jnp.concatenate([a1, a2], axis=0)
    iota_pe = lax.broadcasted_iota(jnp.int32, (NPAIR, E), 1)
    q = (iota_pe == e_pair).astype(jnp.float32)
    c = q
    sh = 1
    while sh < NPAIR:
        c = c + jnp.concatenate(
            [jnp.zeros((sh, E), jnp.float32), c[:-sh, :]], axis=0)
        sh *= 2
    counts = c[NPAIR - 1:NPAIR, :]
    aux_ref[...] = jnp.sum(counts * p_mean, keepdims=True).reshape(1, 1) * (E / N)

    rank = jnp.sum((c - q) * q, axis=1, keepdims=True)
    pcb = jnp.maximum(
        jnp.right_shift(counts.astype(jnp.int32) + (BLK - 1), 7), 1)
    tri = (lax.broadcasted_iota(jnp.int32, (E, E), 0) <
           lax.broadcasted_iota(jnp.int32, (E, E), 1)).astype(jnp.float32)
    offs_b = lax.dot_general(pcb.astype(jnp.float32), tri,
                             (((1,), (0,)), ((), ())),
                             preferred_element_type=jnp.float32)
    pos = rank + jnp.sum(q * (offs_b * float(BLK)), axis=1, keepdims=True)
    pos_ref[...] = pos.astype(jnp.int32)

    bio = lax.broadcasted_iota(jnp.int32, (BE_LEN, E), 0)
    ind = (bio >= offs_b.astype(jnp.int32)).astype(jnp.int32)
    be_ref[...] = jnp.sum(ind, axis=1, keepdims=True) - 1
    be_ref[NB:NB + 1, :] = jnp.sum(pcb, keepdims=True).reshape(1, 1)


def _router(xf, wr):
    return pl.pallas_call(
        _router_body,
        out_shape=[
            jax.ShapeDtypeStruct((NPAIR, 1), jnp.int32),
            jax.ShapeDtypeStruct((NPAIR, 1), jnp.float32),
            jax.ShapeDtypeStruct((BE_LEN, 1), jnp.int32),
            jax.ShapeDtypeStruct((1, 1), jnp.float32),
        ],
        interpret=_INTERPRET,
    )(xf, wr)


_info = plsc.get_sparse_core_info()
_NC, _NS = _info.num_cores, _info.num_subcores
NW = _NC * _NS
PAIRS_W = NPAIR // NW
CH = 64


@functools.partial(
    pl.kernel,
    out_type=jax.ShapeDtypeStruct((P, D), jnp.float32),
    mesh=plsc.VectorSubcoreMesh(core_axis_name="c", subcore_axis_name="s"),
    scratch_types=[
        pltpu.VMEM((CH,), jnp.int32),
        pltpu.VMEM((CH,), jnp.int32),
        pltpu.VMEM((CH, D), jnp.float32),
        pltpu.SemaphoreType.DMA,
    ],
)
def _dispatch(x_hbm, t_hbm, pos_hbm, xs_hbm, t_v, p_v, rows_v, sem):
    wid = lax.axis_index("s") * _NC + lax.axis_index("c")
    for c in range(PAIRS_W // CH):
        base = wid * PAIRS_W + c * CH
        pltpu.sync_copy(t_hbm.at[pl.ds(base, CH)], t_v)
        pltpu.sync_copy(pos_hbm.at[pl.ds(base, CH)], p_v)
        pltpu.async_copy(x_hbm.at[t_v], rows_v, sem).wait()
        pltpu.async_copy(rows_v, xs_hbm.at[p_v], sem).wait()


def _grouped_body(be_ref, xs_ref, wg_ref, wu_ref, wd_ref, ys_ref):
    b = pl.program_id(0)

    @pl.when(b < be_ref[NB])
    def _():
        xb = xs_ref[...]
        g = lax.dot_general(xb, wg_ref[0], (((1,), (1,)), ((), ())),
                            preferred_element_type=jnp.float32)
        u = lax.dot_general(xb, wu_ref[0], (((1,), (1,)), ((), ())),
                            preferred_element_type=jnp.float32)
        h = g * u / (1.0 + jnp.exp(-g))
        ys_ref[...] = lax.dot_general(h, wd_ref[0], (((1,), (1,)), ((), ())),
                                      preferred_element_type=jnp.float32)


def _grouped(be, xs, wg, wu, wd):
    grid_spec = pltpu.PrefetchScalarGridSpec(
        num_scalar_prefetch=1,
        grid=(NB,),
        in_specs=[
            pl.BlockSpec((BLK, D), lambda b, be: (b, 0)),
            pl.BlockSpec((1, F, D), lambda b, be: (be[b], 0, 0)),
            pl.BlockSpec((1, F, D), lambda b, be: (be[b], 0, 0)),
            pl.BlockSpec((1, D, F), lambda b, be: (be[b], 0, 0)),
        ],
        out_specs=pl.BlockSpec((BLK, D), lambda b, be: (b, 0)),
    )
    return pl.pallas_call(
        _grouped_body,
        grid_spec=grid_spec,
        out_shape=jax.ShapeDtypeStruct((P, D), jnp.float32),
        interpret=_INTERPRET,
    )(be, xs, wg, wu, wd)


@functools.partial(
    pl.kernel,
    out_type=jax.ShapeDtypeStruct((NPAIR, D), jnp.float32),
    mesh=plsc.VectorSubcoreMesh(core_axis_name="c", subcore_axis_name="s"),
    scratch_types=[
        pltpu.VMEM((PAIRS_W,), jnp.int32),
        pltpu.VMEM((PAIRS_W, D), jnp.float32),
        pltpu.SemaphoreType.DMA,
    ],
)
def _gather_ys(ys_hbm, pos_hbm, yg_hbm, p_v, rows_v, sem):
    wid = lax.axis_index("s") * _NC + lax.axis_index("c")
    base = wid * PAIRS_W
    pltpu.sync_copy(pos_hbm.at[pl.ds(base, PAIRS_W)], p_v)
    pltpu.async_copy(ys_hbm.at[p_v], rows_v, sem).wait()
    pltpu.sync_copy(rows_v, yg_hbm.at[pl.ds(base, PAIRS_W)])


TB = 256
NTB = N // TB


def _combine_body(x_ref, sg_ref, su_ref, sd_ref, w0_ref, w1_ref,
                  y0_ref, y1_ref, o_ref):
    xb = x_ref[...]
    g = lax.dot_general(xb, sg_ref[...], (((1,), (1,)), ((), ())),
                        preferred_element_type=jnp.float32)
    u = lax.dot_general(xb, su_ref[...], (((1,), (1,)), ((), ())),
                        preferred_element_type=jnp.float32)
    h = g * u / (1.0 + jnp.exp(-g))
    shr = lax.dot_general(h, sd_ref[...], (((1,), (1,)), ((), ())),
                          preferred_element_type=jnp.float32)
    o_ref[...] = shr + w0_ref[...] * y0_ref[...] + w1_ref[...] * y1_ref[...]


def _combine(xf, sg, su, sd, w, yg):
    return pl.pallas_call(
        _combine_body,
        grid=(NTB,),
        in_specs=[
            pl.BlockSpec((TB, D), lambda b: (b, 0)),
            pl.BlockSpec((SF, D), lambda b: (0, 0)),
            pl.BlockSpec((SF, D), lambda b: (0, 0)),
            pl.BlockSpec((D, SF), lambda b: (0, 0)),
            pl.BlockSpec((TB, 1), lambda b: (b, 0)),
            pl.BlockSpec((TB, 1), lambda b: (b + NTB, 0)),
            pl.BlockSpec((TB, D), lambda b: (b, 0)),
            pl.BlockSpec((TB, D), lambda b: (b + NTB, 0)),
        ],
        out_specs=pl.BlockSpec((TB, D), lambda b: (b, 0)),
        out_shape=jax.ShapeDtypeStruct((N, D), jnp.float32),
        interpret=_INTERPRET,
    )(xf, sg, su, sd, w, w, yg, yg)


def kernel(x, Wr, Wg, Wu, Wd, Sg, Su, Sd):
    Bb, Ss, Dm = x.shape
    xf = x.reshape(Bb * Ss, Dm)
    pos2, w2, be2, aux = _router(xf, Wr)
    pos = pos2.reshape(NPAIR)
    be = be2.reshape(BE_LEN)
    t_ids = jnp.tile(jnp.arange(N, dtype=jnp.int32), 2)
    xs = _dispatch(xf, t_ids, pos)
    ys = _grouped(be, xs, Wg, Wu, Wd)
    yg = _gather_ys(ys, pos)
    out = _combine(xf, Sg, Su, Sd, w2, yg)
    return out.reshape(Bb, Ss, Dm), aux.reshape(())


# speedup vs baseline: 5.9793x; 5.9793x over previous
import functools

import jax
import jax.numpy as jnp
from jax import lax
from jax.experimental import pallas as pl
from jax.experimental.pallas import tpu as pltpu
from jax.experimental.pallas import tpu_sc as plsc

N = 2048
E = 64
D = 768
F = 512
SF = 1024
NPAIR = 2 * N
BLK = 128
NB = (NPAIR + E * BLK) // BLK
P = NB * BLK
BE_LEN = 128
NEG = -1e30

_INTERPRET = False


def _router_body(x_ref, wr_ref, pos_ref, w_ref, be_ref, aux_ref):
    x = x_ref[...]
    wr = wr_ref[...]
    logits = lax.dot_general(x, wr, (((1,), (1,)), ((), ())),
                             preferred_element_type=jnp.float32)
    iota_e = lax.broadcasted_iota(jnp.int32, (N, E), 1)
    m1 = jnp.max(logits, axis=1, keepdims=True)
    a1 = jnp.min(jnp.where(logits == m1, iota_e, E), axis=1, keepdims=True)
    masked = jnp.where(iota_e == a1, NEG, logits)
    m2 = jnp.max(masked, axis=1, keepdims=True)
    a2 = jnp.min(jnp.where(masked == m2, iota_e, E), axis=1, keepdims=True)
    s = jnp.exp(m2 - m1)
    w_ref[...] = jnp.concatenate([1.0 / (1.0 + s), s / (1.0 + s)], axis=0)

    sm = jnp.exp(logits - m1)
    sm = sm / jnp.sum(sm, axis=1, keepdims=True)
    p_mean = jnp.sum(sm, axis=0, keepdims=True) * (1.0 / N)

    e_pair = jnp.concatenate([a1, a2], axis=0)
    iota_pe = lax.broadcasted_iota(jnp.int32, (NPAIR, E), 1)
    q = (iota_pe == e_pair).astype(jnp.float32)
    c = q
    sh = 1
    while sh < NPAIR:
        c = c + jnp.concatenate(
            [jnp.zeros((sh, E), jnp.float32), c[:-sh, :]], axis=0)
        sh *= 2
    counts = c[NPAIR - 1:NPAIR, :]
    aux_ref[...] = jnp.sum(counts * p_mean, keepdims=True).reshape(1, 1) * (E / N)

    rank = jnp.sum((c - q) * q, axis=1, keepdims=True)
    pcb = jnp.maximum(
        jnp.right_shift(counts.astype(jnp.int32) + (BLK - 1), 7), 1)
    tri = (lax.broadcasted_iota(jnp.int32, (E, E), 0) <
           lax.broadcasted_iota(jnp.int32, (E, E), 1)).astype(jnp.float32)
    offs_b = lax.dot_general(pcb.astype(jnp.float32), tri,
                             (((1,), (0,)), ((), ())),
                             preferred_element_type=jnp.float32)
    pos = rank + jnp.sum(q * (offs_b * float(BLK)), axis=1, keepdims=True)
    pos_ref[...] = pos.astype(jnp.int32)

    bio = lax.broadcasted_iota(jnp.int32, (BE_LEN, E), 0)
    ind = (bio >= offs_b.astype(jnp.int32)).astype(jnp.int32)
    be_ref[...] = jnp.sum(ind, axis=1, keepdims=True) - 1
    be_ref[NB:NB + 1, :] = jnp.sum(pcb, keepdims=True).reshape(1, 1)


def _router(xf, wr):
    return pl.pallas_call(
        _router_body,
        out_shape=[
            jax.ShapeDtypeStruct((NPAIR, 1), jnp.int32),
            jax.ShapeDtypeStruct((NPAIR, 1), jnp.float32),
            jax.ShapeDtypeStruct((BE_LEN, 1), jnp.int32),
            jax.ShapeDtypeStruct((1, 1), jnp.float32),
        ],
        interpret=_INTERPRET,
    )(xf, wr)


_NC, _NS = 2, 16
NW = _NC * _NS
PAIRS_W = NPAIR // NW
CH = 64


@functools.lru_cache(maxsize=None)
def _dispatch_kernel():
    @functools.partial(
        pl.kernel,
        out_type=jax.ShapeDtypeStruct((P, D), jnp.float32),
        mesh=plsc.VectorSubcoreMesh(core_axis_name="c", subcore_axis_name="s"),
        scratch_types=[
            pltpu.VMEM((CH,), jnp.int32),
            pltpu.VMEM((CH,), jnp.int32),
            pltpu.VMEM((CH, D), jnp.float32),
            pltpu.SemaphoreType.DMA,
        ],
    )
    def body(x_hbm, t_hbm, pos_hbm, xs_hbm, t_v, p_v, rows_v, sem):
        wid = lax.axis_index("s") * _NC + lax.axis_index("c")
        for c in range(PAIRS_W // CH):
            base = wid * PAIRS_W + c * CH
            pltpu.sync_copy(t_hbm.at[pl.ds(base, CH)], t_v)
            pltpu.sync_copy(pos_hbm.at[pl.ds(base, CH)], p_v)
            pltpu.async_copy(x_hbm.at[t_v], rows_v, sem).wait()
            pltpu.async_copy(rows_v, xs_hbm.at[p_v], sem).wait()

    return body


def _dispatch(xf, t_ids, pos):
    return _dispatch_kernel()(xf, t_ids, pos)


def _grouped_body(be_ref, xs_ref, wg_ref, wu_ref, wd_ref, ys_ref):
    b = pl.program_id(0)

    @pl.when(b < be_ref[NB])
    def _():
        xb = xs_ref[...]
        g = lax.dot_general(xb, wg_ref[0], (((1,), (1,)), ((), ())),
                            preferred_element_type=jnp.float32)
        u = lax.dot_general(xb, wu_ref[0], (((1,), (1,)), ((), ())),
                            preferred_element_type=jnp.float32)
        h = g * u / (1.0 + jnp.exp(-g))
        ys_ref[...] = lax.dot_general(h, wd_ref[0], (((1,), (1,)), ((), ())),
                                      preferred_element_type=jnp.float32)


def _grouped(be, xs, wg, wu, wd):
    grid_spec = pltpu.PrefetchScalarGridSpec(
        num_scalar_prefetch=1,
        grid=(NB,),
        in_specs=[
            pl.BlockSpec((BLK, D), lambda b, be: (b, 0)),
            pl.BlockSpec((1, F, D), lambda b, be: (be[b], 0, 0)),
            pl.BlockSpec((1, F, D), lambda b, be: (be[b], 0, 0)),
            pl.BlockSpec((1, D, F), lambda b, be: (be[b], 0, 0)),
        ],
        out_specs=pl.BlockSpec((BLK, D), lambda b, be: (b, 0)),
    )
    return pl.pallas_call(
        _grouped_body,
        grid_spec=grid_spec,
        out_shape=jax.ShapeDtypeStruct((P, D), jnp.float32),
        interpret=_INTERPRET,
    )(be, xs, wg, wu, wd)


@functools.lru_cache(maxsize=None)
def _gather_ys_kernel():
    @functools.partial(
        pl.kernel,
        out_type=jax.ShapeDtypeStruct((NPAIR, D), jnp.float32),
        mesh=plsc.VectorSubcoreMesh(core_axis_name="c", subcore_axis_name="s"),
        scratch_types=[
            pltpu.VMEM((PAIRS_W,), jnp.int32),
            pltpu.VMEM((PAIRS_W, D), jnp.float32),
            pltpu.SemaphoreType.DMA,
        ],
    )
    def body(ys_hbm, pos_hbm, yg_hbm, p_v, rows_v, sem):
        wid = lax.axis_index("s") * _NC + lax.axis_index("c")
        base = wid * PAIRS_W
        pltpu.sync_copy(pos_hbm.at[pl.ds(base, PAIRS_W)], p_v)
        pltpu.async_copy(ys_hbm.at[p_v], rows_v, sem).wait()
        pltpu.sync_copy(rows_v, yg_hbm.at[pl.ds(base, PAIRS_W)])

    return body


def _gather_ys(ys, pos):
    return _gather_ys_kernel()(ys, pos)


TB = 256
NTB = N // TB


def _combine_body(x_ref, sg_ref, su_ref, sd_ref, w0_ref, w1_ref,
                  y0_ref, y1_ref, o_ref):
    xb = x_ref[...]
    g = lax.dot_general(xb, sg_ref[...], (((1,), (1,)), ((), ())),
                        preferred_element_type=jnp.float32)
    u = lax.dot_general(xb, su_ref[...], (((1,), (1,)), ((), ())),
                        preferred_element_type=jnp.float32)
    h = g * u / (1.0 + jnp.exp(-g))
    shr = lax.dot_general(h, sd_ref[...], (((1,), (1,)), ((), ())),
                          preferred_element_type=jnp.float32)
    o_ref[...] = shr + w0_ref[...] * y0_ref[...] + w1_ref[...] * y1_ref[...]


def _combine(xf, sg, su, sd, w, yg):
    return pl.pallas_call(
        _combine_body,
        grid=(NTB,),
        in_specs=[
            pl.BlockSpec((TB, D), lambda b: (b, 0)),
            pl.BlockSpec((SF, D), lambda b: (0, 0)),
            pl.BlockSpec((SF, D), lambda b: (0, 0)),
            pl.BlockSpec((D, SF), lambda b: (0, 0)),
            pl.BlockSpec((TB, 1), lambda b: (b, 0)),
            pl.BlockSpec((TB, 1), lambda b: (b + NTB, 0)),
            pl.BlockSpec((TB, D), lambda b: (b, 0)),
            pl.BlockSpec((TB, D), lambda b: (b + NTB, 0)),
        ],
        out_specs=pl.BlockSpec((TB, D), lambda b: (b, 0)),
        out_shape=jax.ShapeDtypeStruct((N, D), jnp.float32),
        interpret=_INTERPRET,
    )(xf, sg, su, sd, w, w, yg, yg)


def kernel(x, Wr, Wg, Wu, Wd, Sg, Su, Sd):
    Bb, Ss, Dm = x.shape
    xf = x.reshape(Bb * Ss, Dm)
    pos2, w2, be2, aux = _router(xf, Wr)
    pos = pos2.reshape(NPAIR)
    be = be2.reshape(BE_LEN)
    t_ids = jnp.tile(jnp.arange(N, dtype=jnp.int32), 2)
    xs = _dispatch(xf, t_ids, pos)
    ys = _grouped(be, xs, Wg, Wu, Wd)
    yg = _gather_ys(ys, pos)
    out = _combine(xf, Sg, Su, Sd, w2, yg)
    return out.reshape(Bb, Ss, Dm), aux.reshape(())
